# Initial kernel scaffold; baseline (speedup 1.0000x reference)
import jax
import jax.numpy as jnp
from jax.experimental import pallas as pl


def kernel(step_ids, step_embeddings):
    raise NotImplementedError("write your pallas kernel here")


# speedup vs baseline: 1.5992x; 1.5992x over previous
import jax
import jax.numpy as jnp
from jax import lax
from jax.experimental import pallas as pl
from jax.experimental.pallas import tpu as pltpu
from jax.experimental.pallas import tpu_sc as plsc

DIM = 4096
BATCH = 16384
ROWS = 4

_INFO = plsc.get_sparse_core_info()
_NC = _INFO.num_cores
_NS = _INFO.num_subcores
_NW = _NC * _NS
_BPW = BATCH // _NW
_K = 16


def _body(ids_hbm, table_hbm, out_hbm, ids_v, table_v, dsem):
    wid = lax.axis_index("s") * _NC + lax.axis_index("c")
    base = wid * _BPW
    pltpu.sync_copy(ids_hbm.at[pl.ds(base, _BPW)], ids_v)
    pltpu.sync_copy(table_hbm, table_v)

    def group(g, carry):
        gbase = g * _K
        ids_vec = ids_v[pl.ds(gbase, _K)]
        for j in range(_K):
            r = ids_vec[j]
            pltpu.async_copy(table_v.at[r], out_hbm.at[base + gbase + j], dsem)
        for j in range(_K):
            pltpu.make_async_copy(table_v.at[0], out_hbm.at[base + gbase + j], dsem).wait()
        return carry

    lax.fori_loop(0, _BPW // _K, group, 0)


def kernel(step_ids, step_embeddings):
    ids = step_ids.astype(jnp.int32)
    out = pl.kernel(
        _body,
        out_type=jax.ShapeDtypeStruct((BATCH, DIM), jnp.float32),
        mesh=plsc.VectorSubcoreMesh(core_axis_name="c", subcore_axis_name="s"),
        scratch_types=[
            pltpu.VMEM((_BPW,), jnp.int32),
            pltpu.VMEM((ROWS, DIM), jnp.float32),
            pltpu.SemaphoreType.DMA,
        ],
    )(ids, step_embeddings)
    return out[:, None, :]
